# baseline (speedup 1.0000x reference)
import functools

import jax
import jax.numpy as jnp
from jax import lax
from jax.experimental import pallas as pl
from jax.experimental.pallas import tpu as pltpu
from jax.experimental.pallas import tpu_sc as plsc

VOCAB = 1000000
EMBED = 32
BATCH = 16384
HIST = 50

NC = 2
NS = 16
NW = NC * NS
BW = BATCH // NW
CB = 128
KB = BW // CB
NBLK = HIST * KB
NBUF = 10
NGRP = NBLK // NBUF
TE = EMBED // 8
CBP = CB + 1

_mesh = plsc.VectorSubcoreMesh(core_axis_name="c", subcore_axis_name="s")


@functools.partial(
    pl.kernel,
    out_type=jax.ShapeDtypeStruct((HIST, TE, BATCH // CB, 8, CB), jnp.float32),
    mesh=_mesh,
    scratch_types=[
        pltpu.VMEM((HIST, BW), jnp.int32),
        pltpu.VMEM((NBUF, CB, EMBED), jnp.float32),
        pltpu.VMEM((NBUF, TE, 8, CBP), jnp.float32),
        pltpu.SemaphoreType.DMA((NBUF,)),
        pltpu.SemaphoreType.DMA((NBUF,)),
    ],
    compiler_params=pltpu.CompilerParams(
        use_tc_tiling_on_sc=False, needs_layout_passes=False),
)
def _emb_lookup(idx_hbm, table_hbm, out_hbm, idx_v, rows_v, tbuf, gsem, wsem):
    wid = lax.axis_index("s") * NC + lax.axis_index("c")
    pltpu.sync_copy(idx_hbm.at[:, pl.ds(wid * BW, BW)], idx_v)

    iota = lax.iota(jnp.int32, 16)
    te_lo = iota // 8
    re_v = lax.rem(iota, 8)
    te_hi = te_lo + 2

    def g_start(n, b):
        h = n // KB
        k = lax.rem(n, KB)
        pltpu.async_copy(
            table_hbm.at[idx_v.at[h, pl.ds(k * CB, CB)]], rows_v.at[b],
            gsem.at[b])

    def g_wait(b):
        pltpu.make_async_copy(
            table_hbm.at[pl.ds(0, CB)], rows_v.at[b], gsem.at[b]).wait()

    def transpose(b):
        @pl.loop(0, CB, unroll=8)
        def _c(c):
            cv = jnp.full((16,), c, jnp.int32)
            v0 = rows_v[b, c, pl.ds(0, 16)]
            v1 = rows_v[b, c, pl.ds(16, 16)]
            plsc.store_scatter(tbuf.at[b], [te_lo, re_v, cv], v0)
            plsc.store_scatter(tbuf.at[b], [te_hi, re_v, cv], v1)

    def w_start(n, b):
        h = n // KB
        k = lax.rem(n, KB)
        for te in range(TE):
            pltpu.async_copy(
                tbuf.at[b, te, :, pl.ds(0, CB)],
                out_hbm.at[h, te, wid * KB + k], wsem.at[b])

    def w_wait(b):
        for te in range(TE):
            pltpu.make_async_copy(
                tbuf.at[b, te, :, pl.ds(0, CB)], out_hbm.at[0, 0, 0],
                wsem.at[b]).wait()

    for b in range(NBUF):
        g_start(b, b)

    for b in range(NBUF):
        g_wait(b)
        transpose(b)
        g_start(b + NBUF, b)
        w_start(b, b)

    @pl.loop(1, NGRP - 1)
    def _steady(g):
        n0 = g * NBUF
        for b in range(NBUF):
            g_wait(b)
            w_wait(b)
            transpose(b)
            g_start(n0 + b + NBUF, b)
            w_start(n0 + b, b)

    n0 = (NGRP - 1) * NBUF
    for b in range(NBUF):
        g_wait(b)
        w_wait(b)
        transpose(b)
        w_start(n0 + b, b)

    for b in range(NBUF):
        w_wait(b)


def kernel(indices, table):
    out = _emb_lookup(indices.T.astype(jnp.int32), table)
    return out.transpose(2, 4, 0, 1, 3).reshape(BATCH, HIST, EMBED)


# speedup vs baseline: 1.0109x; 1.0109x over previous
import functools

import jax
import jax.numpy as jnp
from jax import lax
from jax.experimental import pallas as pl
from jax.experimental.pallas import tpu as pltpu
from jax.experimental.pallas import tpu_sc as plsc

VOCAB = 1000000
EMBED = 32
BATCH = 16384
HIST = 50

NC = 2
NS = 16
NW = NC * NS
BW = BATCH // NW
CB = 128
KB = BW // CB
NBLK = HIST * KB
NBUF = 8
NGRP = NBLK // NBUF
TE = EMBED // 8
CBP = CB + 1

_mesh = plsc.VectorSubcoreMesh(core_axis_name="c", subcore_axis_name="s")


@functools.partial(
    pl.kernel,
    out_type=jax.ShapeDtypeStruct((HIST, TE, BATCH // CB, 8, CB), jnp.float32),
    mesh=_mesh,
    scratch_types=[
        pltpu.VMEM((HIST, BW), jnp.int32),
        pltpu.VMEM((NBUF, CB, EMBED), jnp.float32),
        pltpu.VMEM((NBUF, TE, 8, CBP), jnp.float32),
        pltpu.SemaphoreType.DMA((NBUF,)),
        pltpu.SemaphoreType.DMA((NBUF,)),
    ],
    compiler_params=pltpu.CompilerParams(
        use_tc_tiling_on_sc=False, needs_layout_passes=False,
        disable_semaphore_checks=True),
)
def _emb_lookup(idx_hbm, table_hbm, out_hbm, idx_v, rows_v, tbuf, gsem, wsem):
    wid = lax.axis_index("s") * NC + lax.axis_index("c")
    pltpu.sync_copy(idx_hbm.at[:, pl.ds(wid * BW, BW)], idx_v)

    iota = lax.iota(jnp.int32, 16)
    te_lo = iota // 8
    re_v = lax.rem(iota, 8)
    te_hi = te_lo + 2

    def g_start(n, b):
        h = n // KB
        k = lax.rem(n, KB)
        pltpu.async_copy(
            table_hbm.at[idx_v.at[h, pl.ds(k * CB, CB)]], rows_v.at[b],
            gsem.at[b])

    def g_wait(b):
        pltpu.make_async_copy(
            table_hbm.at[pl.ds(0, CB)], rows_v.at[b], gsem.at[b]).wait()

    def transpose(b):
        @pl.loop(0, CB, unroll=8)
        def _c(c):
            cv = jnp.full((16,), c, jnp.int32)
            v0 = rows_v[b, c, pl.ds(0, 16)]
            v1 = rows_v[b, c, pl.ds(16, 16)]
            plsc.store_scatter(tbuf.at[b], [te_lo, re_v, cv], v0)
            plsc.store_scatter(tbuf.at[b], [te_hi, re_v, cv], v1)

    def w_start(n, b):
        h = n // KB
        k = lax.rem(n, KB)
        for te in range(TE):
            pltpu.async_copy(
                tbuf.at[b, te, :, pl.ds(0, CB)],
                out_hbm.at[h, te, wid * KB + k], wsem.at[b])

    def w_wait(b):
        for te in range(TE):
            pltpu.make_async_copy(
                tbuf.at[b, te, :, pl.ds(0, CB)], out_hbm.at[0, 0, 0],
                wsem.at[b]).wait()

    for b in range(NBUF):
        g_start(b, b)

    for b in range(NBUF):
        g_wait(b)
        transpose(b)
        g_start(b + NBUF, b)
        w_start(b, b)

    @pl.loop(1, NGRP - 1)
    def _steady(g):
        n0 = g * NBUF
        for b in range(NBUF):
            g_wait(b)
            w_wait(b)
            transpose(b)
            g_start(n0 + b + NBUF, b)
            w_start(n0 + b, b)

    n0 = (NGRP - 1) * NBUF
    for b in range(NBUF):
        g_wait(b)
        w_wait(b)
        transpose(b)
        w_start(n0 + b, b)

    for b in range(NBUF):
        w_wait(b)


def kernel(indices, table):
    out = _emb_lookup(indices.T.astype(jnp.int32), table)
    return out.transpose(2, 4, 0, 1, 3).reshape(BATCH, HIST, EMBED)


# speedup vs baseline: 1.0162x; 1.0052x over previous
import functools

import jax
import jax.numpy as jnp
from jax import lax
from jax.experimental import pallas as pl
from jax.experimental.pallas import tpu as pltpu
from jax.experimental.pallas import tpu_sc as plsc

VOCAB = 1000000
EMBED = 32
BATCH = 16384
HIST = 50

NC = 2
NS = 16
NW = NC * NS
BW = BATCH // NW
CB = 128
KB = BW // CB
NBLK = HIST * KB
NBUF = 8
NGRP = NBLK // NBUF
TE = EMBED // 8
CBP = CB + 1

_mesh = plsc.VectorSubcoreMesh(core_axis_name="c", subcore_axis_name="s")


@functools.partial(
    pl.kernel,
    out_type=jax.ShapeDtypeStruct((HIST, TE, BATCH // CB, 8, CB), jnp.float32),
    mesh=_mesh,
    scratch_types=[
        pltpu.VMEM((HIST, BW), jnp.int32),
        pltpu.VMEM((NBUF, CB, EMBED), jnp.float32),
        pltpu.VMEM((NBUF, TE, 8, CBP), jnp.float32),
        pltpu.SemaphoreType.DMA((NBUF,)),
        pltpu.SemaphoreType.DMA((NBUF,)),
    ],
    compiler_params=pltpu.CompilerParams(
        use_tc_tiling_on_sc=False, needs_layout_passes=False),
)
def _emb_lookup(idx_hbm, table_hbm, out_hbm, idx_v, rows_v, tbuf, gsem, wsem):
    wid = lax.axis_index("s") * NC + lax.axis_index("c")
    pltpu.sync_copy(idx_hbm.at[:, pl.ds(wid * BW, BW)], idx_v)

    iota = lax.iota(jnp.int32, 16)
    te_lo = iota // 8
    re_v = lax.rem(iota, 8)
    te_hi = te_lo + 2

    def g_start(n, b):
        h = n // KB
        k = lax.rem(n, KB)
        pltpu.async_copy(
            table_hbm.at[idx_v.at[h, pl.ds(k * CB, CB)]], rows_v.at[b],
            gsem.at[b])

    def g_wait(b):
        pltpu.make_async_copy(
            table_hbm.at[pl.ds(0, CB)], rows_v.at[b], gsem.at[b]).wait()

    def transpose(b):
        @pl.loop(0, CB, unroll=8)
        def _c(c):
            cv = jnp.full((16,), c, jnp.int32)
            v0 = rows_v[b, c, pl.ds(0, 16)]
            v1 = rows_v[b, c, pl.ds(16, 16)]
            plsc.store_scatter(tbuf.at[b], [te_lo, re_v, cv], v0)
            plsc.store_scatter(tbuf.at[b], [te_hi, re_v, cv], v1)

    def w_start(n, b):
        h = n // KB
        k = lax.rem(n, KB)
        for te in range(TE):
            pltpu.async_copy(
                tbuf.at[b, te, :, pl.ds(0, CB)],
                out_hbm.at[h, te, wid * KB + k], wsem.at[b])

    def w_wait(b):
        for te in range(TE):
            pltpu.make_async_copy(
                tbuf.at[b, te, :, pl.ds(0, CB)], out_hbm.at[0, 0, 0],
                wsem.at[b]).wait()

    for b in range(NBUF):
        g_start(b, b)

    for b in range(NBUF):
        g_wait(b)
        transpose(b)
        g_start(b + NBUF, b)
        w_start(b, b)

    @pl.loop(1, NGRP - 1)
    def _steady(g):
        n0 = g * NBUF
        for b in range(NBUF):
            g_wait(b)
            w_wait(b)
            transpose(b)
            g_start(n0 + b + NBUF, b)
            w_start(n0 + b, b)

    n0 = (NGRP - 1) * NBUF
    for b in range(NBUF):
        g_wait(b)
        w_wait(b)
        transpose(b)
        w_start(n0 + b, b)

    for b in range(NBUF):
        w_wait(b)


def kernel(indices, table):
    out = _emb_lookup(indices.T.astype(jnp.int32), table)
    return out.transpose(2, 4, 0, 1, 3).reshape(BATCH, HIST, EMBED)
